# baseline (speedup 1.0000x reference)
import functools

import jax
import jax.numpy as jnp
from jax import lax
from jax.experimental import pallas as pl
from jax.experimental.pallas import tpu as pltpu
from jax.experimental.pallas import tpu_sc as plsc


def _flatten_tables(tables_t, C, V, D):
    VC = 4096
    n_vc = pl.cdiv(V, VC)

    def body(in_ref, out_ref):
        eye = jnp.eye(D, dtype=jnp.float32)
        res = lax.dot_general(
            in_ref[0], eye, (((0,), (0,)), ((), ())),
            preferred_element_type=jnp.float32)
        out_ref[0, :, :D] = res.astype(jnp.bfloat16)

    return pl.pallas_call(
        body,
        grid=(C, n_vc),
        in_specs=[pl.BlockSpec((1, D, VC), lambda c, v: (c, 0, v))],
        out_specs=pl.BlockSpec((1, VC, 2 * D), lambda c, v: (c, v, 0)),
        out_shape=jax.ShapeDtypeStruct((C, V, 2 * D), jnp.bfloat16),
        compiler_params=pltpu.CompilerParams(
            dimension_semantics=("arbitrary", "arbitrary"),
        ),
    )(tables_t)


def kernel(x, continuous_x_res, tables):
    B, C = x.shape
    _, NCONT, D = continuous_x_res.shape
    V = tables.shape[1]
    OUT_C = C + NCONT

    tables_t = tables.transpose(0, 2, 1)
    tables_pad = _flatten_tables(tables_t, C, V, D)
    tables_flat = tables_pad.reshape(C * V, 2 * D)

    cont_bf16 = continuous_x_res.astype(jnp.bfloat16)

    flat_idx = x + (jnp.arange(C, dtype=jnp.int32) * V)[None, :]

    NC, NS = 2, 16
    NW = NC * NS
    b_per_w = B // NW
    BK = 16
    steps = b_per_w // BK

    mesh = plsc.VectorSubcoreMesh(core_axis_name="c", subcore_axis_name="s")

    @functools.partial(
        pl.kernel,
        mesh=mesh,
        out_type=jax.ShapeDtypeStruct((B, OUT_C, D), jnp.bfloat16),
        compiler_params=pltpu.CompilerParams(use_tc_tiling_on_sc=False),
        scratch_types=[
            pltpu.VMEM((BK, C), jnp.int32),
            pltpu.VMEM((BK, OUT_C, 2 * D), jnp.bfloat16),
            pltpu.SemaphoreType.DMA,
            pltpu.SemaphoreType.DMA,
        ],
    )
    def gather_concat(tab_hbm, idx_hbm, cont_hbm, out_hbm, idx_v, vbuf,
                      sem_g, sem_c):
        wid = lax.axis_index("s") * NC + lax.axis_index("c")
        base = wid * b_per_w

        @pl.loop(0, steps)
        def _(t):
            row0 = base + t * BK
            pltpu.sync_copy(idx_hbm.at[pl.ds(row0, BK)], idx_v)
            cont_cp = pltpu.async_copy(
                cont_hbm.at[pl.ds(row0, BK)],
                vbuf.at[:, pl.ds(C, NCONT), pl.ds(0, D)],
                sem_c,
            )
            gathers = []
            for j in range(BK):
                gathers.append(pltpu.async_copy(
                    tab_hbm.at[idx_v.at[j]],
                    vbuf.at[j, pl.ds(0, C)],
                    sem_g,
                ))
            for cp in gathers:
                cp.wait()
            cont_cp.wait()
            pltpu.sync_copy(vbuf.at[:, :, pl.ds(0, D)],
                            out_hbm.at[pl.ds(row0, BK)])

    out_bf16 = gather_concat(tables_flat, flat_idx, cont_bf16)
    return out_bf16.astype(jnp.float32)


# speedup vs baseline: 1.5280x; 1.5280x over previous
import functools

import jax
import jax.numpy as jnp
from jax import lax
from jax.experimental import pallas as pl
from jax.experimental.pallas import tpu as pltpu
from jax.experimental.pallas import tpu_sc as plsc


def kernel(x, continuous_x_res, tables):
    B, C = x.shape
    _, NCONT, D = continuous_x_res.shape
    V = tables.shape[1]
    OUT_C = C + NCONT

    xT = x.T

    NC, NS = 2, 16
    NW = NC * NS
    b_per_w = B // NW
    BK = 64
    steps = b_per_w // BK

    mesh = plsc.VectorSubcoreMesh(core_axis_name="c", subcore_axis_name="s")

    @functools.partial(
        pl.kernel,
        mesh=mesh,
        out_type=jax.ShapeDtypeStruct((B, OUT_C, D), jnp.float32),
        compiler_params=pltpu.CompilerParams(use_tc_tiling_on_sc=False),
        scratch_types=[
            pltpu.VMEM((C, BK), jnp.int32),
            pltpu.VMEM((C, BK, D), jnp.float32),
            pltpu.SemaphoreType.DMA,
            pltpu.SemaphoreType.DMA,
        ],
    )
    def k(tab3_hbm, idx_hbm, cont_hbm, out_hbm, idx_v, gbuf, sem_g, sem_c):
        wid = lax.axis_index("s") * NC + lax.axis_index("c")
        base = wid * b_per_w

        cont_cp = pltpu.async_copy(
            cont_hbm.at[pl.ds(base, b_per_w)],
            out_hbm.at[pl.ds(base, b_per_w), pl.ds(C, NCONT)],
            sem_c,
        )

        @pl.loop(0, steps)
        def _(t):
            row0 = base + t * BK
            pltpu.sync_copy(idx_hbm.at[:, pl.ds(row0, BK)], idx_v)
            gathers = []
            for c in range(C):
                gathers.append(pltpu.async_copy(
                    tab3_hbm.at[c].at[idx_v.at[c]],
                    gbuf.at[c],
                    sem_g,
                ))
            for cp in gathers:
                cp.wait()
            writes = []
            for c in range(C):
                writes.append(pltpu.async_copy(
                    gbuf.at[c],
                    out_hbm.at[pl.ds(row0, BK), c],
                    sem_g,
                ))
            for cp in writes:
                cp.wait()

        cont_cp.wait()

    return k(tables, xT, continuous_x_res)


# speedup vs baseline: 1.9649x; 1.2859x over previous
import functools

import jax
import jax.numpy as jnp
from jax import lax
from jax.experimental import pallas as pl
from jax.experimental.pallas import tpu as pltpu
from jax.experimental.pallas import tpu_sc as plsc


def kernel(x, continuous_x_res, tables):
    B, C = x.shape
    _, NCONT, D = continuous_x_res.shape
    V = tables.shape[1]
    OUT_C = C + NCONT

    tables_flat = tables.reshape(C * V, D)
    flat_idx = x + (jnp.arange(C, dtype=jnp.int32) * V)[None, :]

    NC, NS = 2, 16
    NW = NC * NS
    b_per_w = B // NW
    BK = 16
    steps = b_per_w // BK

    mesh = plsc.VectorSubcoreMesh(core_axis_name="c", subcore_axis_name="s")

    @functools.partial(
        pl.kernel,
        mesh=mesh,
        out_type=jax.ShapeDtypeStruct((B, OUT_C, D), jnp.float32),
        compiler_params=pltpu.CompilerParams(use_tc_tiling_on_sc=False),
        scratch_types=[
            pltpu.VMEM((BK, C), jnp.int32),
            pltpu.VMEM((BK, OUT_C, D), jnp.float32),
            pltpu.SemaphoreType.DMA,
            pltpu.SemaphoreType.DMA,
        ],
    )
    def k(tab_hbm, idx_hbm, cont_hbm, out_hbm, idx_v, vbuf, sem_g, sem_c):
        wid = lax.axis_index("s") * NC + lax.axis_index("c")
        base = wid * b_per_w

        @pl.loop(0, steps)
        def _(t):
            row0 = base + t * BK
            pltpu.sync_copy(idx_hbm.at[pl.ds(row0, BK)], idx_v)
            cont_cp = pltpu.async_copy(
                cont_hbm.at[pl.ds(row0, BK)],
                vbuf.at[:, pl.ds(C, NCONT)],
                sem_c,
            )
            gathers = []
            for j in range(BK):
                gathers.append(pltpu.async_copy(
                    tab_hbm.at[idx_v.at[j]],
                    vbuf.at[j, pl.ds(0, C)],
                    sem_g,
                ))
            for cp in gathers:
                cp.wait()
            cont_cp.wait()
            pltpu.sync_copy(vbuf, out_hbm.at[pl.ds(row0, BK)])

    return k(tables_flat, flat_idx, continuous_x_res)
